# baseline (speedup 1.0000x reference)
import functools

import jax
import jax.numpy as jnp
from jax import lax
from jax.experimental import pallas as pl
from jax.experimental.pallas import tpu as pltpu
from jax.experimental.pallas import tpu_sc as plsc

B, C, H, W = 2, 192, 384, 384
N = 8192
K = 9
HID = 128
HWp = H * W
BHW = B * HWp
BN = B * N
NC, NS, L = 2, 16, 16
NW = NC * NS
PPT = BN // NW
J = 4 * K
CV = C // L
CP = 256

_MESH = plsc.VectorSubcoreMesh(core_axis_name="c", subcore_axis_name="s",
                               num_cores=NC, num_subcores=NS)

_TROWS = 8
C2 = 2 * C


def _transpose_body(f_ref, n_ref, t_ref):
    for s in range(_TROWS):
        left = f_ref[0, :, s, :].T
        if s + 1 < _TROWS:
            nxt0 = f_ref[0, :, s + 1, 0]
        else:
            nxt0 = n_ref[0, :, 0, 0]
        right = jnp.concatenate([left[1:, :], nxt0[None, :]], axis=0)
        t_ref[s * W:(s + 1) * W, :] = jnp.concatenate([left, right], axis=1)


def _make_table(feat_map):
    nh = H // _TROWS
    return pl.pallas_call(
        _transpose_body,
        grid=(B, nh),
        in_specs=[
            pl.BlockSpec((1, C, _TROWS, W), lambda b, h: (b, 0, h, 0)),
            pl.BlockSpec((1, C, _TROWS, W),
                         lambda b, h: (b, 0,
                                       jnp.minimum(h + 1, H // _TROWS - 1),
                                       0)),
        ],
        out_specs=pl.BlockSpec((_TROWS * W, C2),
                               lambda b, h: (b * (H // _TROWS) + h, 0)),
        out_shape=jax.ShapeDtypeStruct((BHW, C2), jnp.float32),
    )(feat_map, feat_map)


_AG = 16
_ACH = PPT // _AG
_ANB = 2


def _anchor_body(table, xs, ys, out, xv, yv, idxv, wbuf, rows, outv, sems):
    wid = lax.axis_index("s") * NC + lax.axis_index("c")
    base_pt = wid * PPT
    pltpu.sync_copy(xs.at[pl.ds(base_pt, PPT)], xv)
    pltpu.sync_copy(ys.at[pl.ds(base_pt, PPT)], yv)

    def weights(i):
        off = i * _AG
        x = xv[pl.ds(off, L)]
        y = yv[pl.ds(off, L)]
        gx = jnp.clip((x + 1.0) * (0.5 * (W - 1)), 0.0, float(W - 1))
        gy = jnp.clip((y + 1.0) * (0.5 * (H - 1)), 0.0, float(H - 1))
        x0 = gx.astype(jnp.int32)
        y0 = gy.astype(jnp.int32)
        fx1 = gx - x0.astype(jnp.float32)
        fy1 = gy - y0.astype(jnp.float32)
        return x0, y0, fx1, fy1

    def issue(i, b):
        x0, y0, _, _ = weights(i)
        off = i * _AG
        pt = base_pt + off + lax.iota(jnp.int32, L)
        bb = jnp.where(pt >= N, HWp, 0).astype(jnp.int32)
        p00 = bb + y0 * W + x0
        idxv[b, pl.ds(0, L)] = p00
        idxv[b, pl.ds(L, L)] = jnp.minimum(p00 + W, BHW - 1)
        pltpu.async_copy(table.at[idxv.at[b]], rows[b], sems[b])

    for b in range(_ANB):
        issue(b, b)

    def chunk2(i2, carry):
        for sb in range(_ANB):
            i = i2 * _ANB + sb
            _, _, fx1, fy1 = weights(i)
            fx0 = 1.0 - fx1
            fy0 = 1.0 - fy1
            wv0 = fy0 * fx0
            wv1 = fy0 * fx1
            wv2 = fy1 * fx0
            wv3 = fy1 * fx1
            pltpu.make_async_copy(table.at[idxv.at[sb]], rows[sb],
                                  sems[sb]).wait()
            for g in range(_AG):
                w0 = wv0[g]
                w1 = wv1[g]
                w2 = wv2[g]
                w3 = wv3[g]
                for c in range(CV):
                    sl = pl.ds(c * L, L)
                    sr = pl.ds(C + c * L, L)
                    outv[pl.ds((sb * _AG + g) * C + c * L, L)] = (
                        rows[sb][g, sl] * w0 + rows[sb][g, sr] * w1
                        + rows[sb][L + g, sl] * w2 + rows[sb][L + g, sr] * w3)

            @pl.when(i + _ANB < _ACH)
            def _():
                issue(i + _ANB, sb)

        pltpu.sync_copy(outv,
                        out.at[pl.ds((base_pt + i2 * _ANB * _AG) * C,
                                     _ANB * _AG * C)])
        return carry

    lax.fori_loop(0, _ACH // _ANB, chunk2, 0)


_anchor_call = functools.partial(
    pl.kernel,
    out_type=jax.ShapeDtypeStruct((BN * C,), jnp.float32),
    mesh=_MESH,
    compiler_params=pltpu.CompilerParams(needs_layout_passes=False),
    scratch_types=[
        pltpu.VMEM((PPT,), jnp.float32),
        pltpu.VMEM((PPT,), jnp.float32),
        pltpu.VMEM((_ANB, 2 * L), jnp.int32),
        pltpu.VMEM((4 * L,), jnp.float32),
        [pltpu.VMEM((2 * L, C2), jnp.float32) for _ in range(_ANB)],
        pltpu.VMEM((_ANB * _AG * C,), jnp.float32),
        [pltpu.SemaphoreType.DMA for _ in range(_ANB)],
    ],
)


_BLKP = 1024


def _leaky(x):
    return jnp.where(x >= 0, x, 0.2 * x)


def _softplus(x):
    return jnp.maximum(x, 0.0) + jnp.log1p(jnp.exp(-jnp.abs(x)))


def _dot(a, b):
    return lax.dot_general(a, b, (((1,), (0,)), ((), ())),
                           preferred_element_type=jnp.float32,
                           precision=lax.Precision.HIGHEST)


def _mlp_body(fa_ref, co_ref, ce_ref, W1_ref, b1_ref, W2_ref, b2_ref,
              W3_ref, b3_ref, idx_ref, w_ref):
    fa = fa_ref[...]
    co = co_ref[...]
    ce = ce_ref[...]
    W1 = W1_ref[...]
    h = _dot(fa, W1[:C, :])
    h += co[:, 0:1] * W1[C:C + 1, :] + co[:, 1:2] * W1[C + 1:C + 2, :]
    h += ce[:, 0:1] * W1[C + 2:C + 3, :] + ce[:, 1:2] * W1[C + 3:C + 4, :]
    h += b1_ref[...]
    h = _leaky(h)
    h = _leaky(h + _dot(h, W2_ref[...]) + b2_ref[...])
    o = _dot(h, W3_ref[...]) + b3_ref[...]

    r = jnp.clip(_softplus(o[:, 0:1]) + 0.1, 0.1, 4.0)
    sigma = jnp.clip(_softplus(o[:, 1:2]) + 0.5, 0.5, 6.0)
    res18 = jnp.tanh(o[:, 2:2 + 2 * K]) * 0.5
    graw = o[:, 2 + 2 * K:2 + 3 * K]
    gate = 1.0 / (1.0 + jnp.exp(-graw))

    ii = lax.broadcasted_iota(jnp.int32, (1, 2 * K), 1)
    kk = ii // 2
    base18 = jnp.where(ii % 2 == 0, kk % 3 - 1, kk // 3 - 1).astype(jnp.float32)
    off18 = r * base18 + res18

    rows18 = lax.broadcasted_iota(jnp.int32, (2 * K, K), 0)
    cols18 = lax.broadcasted_iota(jnp.int32, (2 * K, K), 1)
    Mx = (rows18 == 2 * cols18).astype(jnp.float32)
    My = (rows18 == 2 * cols18 + 1).astype(jnp.float32)
    offx = _dot(off18, Mx)
    offy = _dot(off18, My)
    d2 = _dot(off18 * off18, Mx + My)

    sig_eff = sigma * 2.0
    w_geo = jnp.exp(-0.5 * d2 / (sig_eff * sig_eff + 1e-08))
    wk = w_geo * gate
    wn = wk / (jnp.sum(wk, axis=1, keepdims=True) + 1e-08)

    sx = 2.0 / (W - 1)
    sy = 2.0 / (H - 1)
    gx = jnp.clip((co[:, 0:1] + offx * sx + 1.0) * (0.5 * (W - 1)),
                  0.0, float(W - 1))
    gy = jnp.clip((co[:, 1:2] + offy * sy + 1.0) * (0.5 * (H - 1)),
                  0.0, float(H - 1))
    x0 = gx.astype(jnp.int32)
    y0 = gy.astype(jnp.int32)
    fx1 = gx - x0.astype(jnp.float32)
    fy1 = gy - y0.astype(jnp.float32)
    fx0 = 1.0 - fx1
    fy0 = 1.0 - fy1

    p = (pl.program_id(0) * _BLKP
         + lax.broadcasted_iota(jnp.int32, (_BLKP, 1), 0))
    bb = jnp.where(p >= N, HWp, 0).astype(jnp.int32)
    i00 = bb + y0 * W + x0
    i10 = jnp.minimum(i00 + W, BHW - 1)
    idx_ref[...] = jnp.concatenate([i00, i10], axis=1)
    w_ref[...] = jnp.concatenate([wn * fy0 * fx0, wn * fy0 * fx1,
                                  wn * fy1 * fx0, wn * fy1 * fx1], axis=1)


def _mlp_prep(f_anchor, co, ce, W1, b1, W2, b2, W3, b3):
    full = lambda i, j=None: (0, 0)
    grid = (BN // _BLKP,)
    return pl.pallas_call(
        _mlp_body,
        grid=grid,
        in_specs=[
            pl.BlockSpec((_BLKP, C), lambda i: (i, 0)),
            pl.BlockSpec((_BLKP, 2), lambda i: (i, 0)),
            pl.BlockSpec((_BLKP, 2), lambda i: (i, 0)),
            pl.BlockSpec((C + 4, HID), lambda i: (0, 0)),
            pl.BlockSpec((1, HID), lambda i: (0, 0)),
            pl.BlockSpec((HID, HID), lambda i: (0, 0)),
            pl.BlockSpec((1, HID), lambda i: (0, 0)),
            pl.BlockSpec((HID, 2 + 3 * K), lambda i: (0, 0)),
            pl.BlockSpec((1, 2 + 3 * K), lambda i: (0, 0)),
        ],
        out_specs=[
            pl.BlockSpec((_BLKP, 2 * K), lambda i: (i, 0)),
            pl.BlockSpec((_BLKP, J), lambda i: (i, 0)),
        ],
        out_shape=[
            jax.ShapeDtypeStruct((BN, 2 * K), jnp.int32),
            jax.ShapeDtypeStruct((BN, J), jnp.float32),
        ],
    )(f_anchor, co, ce, W1, b1, W2, b2, W3, b3)


_DG = 4
_CHN = PPT // _DG
_NBUF = 2
_JR = 2 * K


def _deform_body(table, idx_hbm, w_hbm, out, idxv, wtv, outv, rows, sems):
    wid = lax.axis_index("s") * NC + lax.axis_index("c")
    base_pt = wid * PPT
    pltpu.sync_copy(idx_hbm.at[pl.ds(wid * _CHN, _CHN)], idxv)
    pltpu.sync_copy(w_hbm.at[pl.ds(base_pt * J, PPT * J)],
                    wtv.at[pl.ds(0, PPT * J)])

    for b in range(_NBUF):
        pltpu.async_copy(table.at[idxv.at[b]], rows[b], sems[b])

    def chunk(ci, carry):
        for b in range(_NBUF):
            cc = ci * _NBUF + b
            pltpu.make_async_copy(table.at[idxv.at[0]], rows[b],
                                  sems[b]).wait()
            for g in range(_DG):
                wbase = jnp.full((L,), (cc * _DG + g) * J, jnp.int32)
                accs = [None] * CV
                for k in range(K):
                    w00 = plsc.load_gather(wtv, [wbase + k])
                    w01 = plsc.load_gather(wtv, [wbase + (K + k)])
                    w10 = plsc.load_gather(wtv, [wbase + (2 * K + k)])
                    w11 = plsc.load_gather(wtv, [wbase + (3 * K + k)])
                    rA = g * _JR + k
                    rB = g * _JR + K + k
                    for c in range(CV):
                        sl = pl.ds(c * L, L)
                        sr = pl.ds(C + c * L, L)
                        t = (rows[b][rA, sl] * w00 + rows[b][rA, sr] * w01
                             + rows[b][rB, sl] * w10 + rows[b][rB, sr] * w11)
                        accs[c] = t if k == 0 else accs[c] + t
                for c in range(CV):
                    outv[pl.ds(g * C + c * L, L)] = accs[c]
            nxt = cc + _NBUF

            @pl.when(nxt < _CHN)
            def _():
                pltpu.async_copy(table.at[idxv.at[nxt]], rows[b], sems[b])

            pltpu.sync_copy(outv, out.at[pl.ds((base_pt + cc * _DG) * C,
                                               _DG * C)])
        return carry

    lax.fori_loop(0, _CHN // _NBUF, chunk, 0)


_deform_call = functools.partial(
    pl.kernel,
    out_type=jax.ShapeDtypeStruct((BN * C,), jnp.float32),
    mesh=_MESH,
    compiler_params=pltpu.CompilerParams(needs_layout_passes=False),
    scratch_types=[
        pltpu.VMEM((_CHN, _DG * _JR), jnp.int32),
        pltpu.VMEM((PPT * J + L,), jnp.float32),
        pltpu.VMEM((_DG * C,), jnp.float32),
        [pltpu.VMEM((_DG * _JR, C2), jnp.float32) for _ in range(_NBUF)],
        [pltpu.SemaphoreType.DMA for _ in range(_NBUF)],
    ],
)


def kernel(feat_map, coords_2d, cell_2d, W1, b1, W2, b2, W3, b3):
    table = _make_table(feat_map)

    co = coords_2d.reshape(BN, 2)
    ce = cell_2d.reshape(BN, 2)
    xs = co[:, 0].reshape(BN)
    ys = co[:, 1].reshape(BN)

    f_anchor = _anchor_call(_anchor_body)(table, xs, ys).reshape(BN, C)

    idx_all, w_all = _mlp_prep(f_anchor, co, ce, W1,
                               b1.reshape(1, HID), W2, b2.reshape(1, HID),
                               W3, b3.reshape(1, 2 + 3 * K))

    out = _deform_call(_deform_body)(table,
                                     idx_all.reshape(BN // _DG, _DG * _JR),
                                     w_all.reshape(BN * J))
    return out.reshape(B, N, C)


# speedup vs baseline: 1.1165x; 1.1165x over previous
import functools

import jax
import jax.numpy as jnp
from jax import lax
from jax.experimental import pallas as pl
from jax.experimental.pallas import tpu as pltpu
from jax.experimental.pallas import tpu_sc as plsc

B, C, H, W = 2, 192, 384, 384
N = 8192
K = 9
HID = 128
HWp = H * W
BHW = B * HWp
BN = B * N
NC, NS, L = 2, 16, 16
NW = NC * NS
PPT = BN // NW
J = 4 * K
CV = C // L
CP = 256

_MESH = plsc.VectorSubcoreMesh(core_axis_name="c", subcore_axis_name="s",
                               num_cores=NC, num_subcores=NS)

_TROWS = 8
C2 = 2 * C


def _transpose_body(f_ref, n_ref, t_ref):
    for s in range(_TROWS):
        left = f_ref[0, :, s, :].T
        if s + 1 < _TROWS:
            nxt0 = f_ref[0, :, s + 1, 0]
        else:
            nxt0 = n_ref[0, :, 0, 0]
        right = jnp.concatenate([left[1:, :], nxt0[None, :]], axis=0)
        t_ref[s * W:(s + 1) * W, :] = jnp.concatenate([left, right], axis=1)


def _make_table(feat_map):
    nh = H // _TROWS
    return pl.pallas_call(
        _transpose_body,
        grid=(B, nh),
        in_specs=[
            pl.BlockSpec((1, C, _TROWS, W), lambda b, h: (b, 0, h, 0)),
            pl.BlockSpec((1, C, _TROWS, 128),
                         lambda b, h: (b, 0,
                                       jnp.minimum(h + 1, H // _TROWS - 1),
                                       0)),
        ],
        out_specs=pl.BlockSpec((_TROWS * W, C2),
                               lambda b, h: (b * (H // _TROWS) + h, 0)),
        out_shape=jax.ShapeDtypeStruct((BHW, C2), jnp.float32),
    )(feat_map, feat_map)


_AG = 16
_ACH = PPT // _AG
_ANB = 2


def _anchor_body(table, xs, ys, out, xv, yv, idxv, wbuf, rows, outv, sems):
    wid = lax.axis_index("s") * NC + lax.axis_index("c")
    base_pt = wid * PPT
    pltpu.sync_copy(xs.at[pl.ds(base_pt, PPT)], xv)
    pltpu.sync_copy(ys.at[pl.ds(base_pt, PPT)], yv)

    def weights(i):
        off = i * _AG
        x = xv[pl.ds(off, L)]
        y = yv[pl.ds(off, L)]
        gx = jnp.clip((x + 1.0) * (0.5 * (W - 1)), 0.0, float(W - 1))
        gy = jnp.clip((y + 1.0) * (0.5 * (H - 1)), 0.0, float(H - 1))
        x0 = gx.astype(jnp.int32)
        y0 = gy.astype(jnp.int32)
        fx1 = gx - x0.astype(jnp.float32)
        fy1 = gy - y0.astype(jnp.float32)
        return x0, y0, fx1, fy1

    def issue(i, b):
        x0, y0, _, _ = weights(i)
        off = i * _AG
        pt = base_pt + off + lax.iota(jnp.int32, L)
        bb = jnp.where(pt >= N, HWp, 0).astype(jnp.int32)
        p00 = bb + y0 * W + x0
        idxv[b, pl.ds(0, L)] = p00
        idxv[b, pl.ds(L, L)] = jnp.minimum(p00 + W, BHW - 1)
        pltpu.async_copy(table.at[idxv.at[b]], rows[b], sems[b])

    for b in range(_ANB):
        issue(b, b)

    def chunk2(i2, carry):
        for sb in range(_ANB):
            i = i2 * _ANB + sb
            _, _, fx1, fy1 = weights(i)
            fx0 = 1.0 - fx1
            fy0 = 1.0 - fy1
            wv0 = fy0 * fx0
            wv1 = fy0 * fx1
            wv2 = fy1 * fx0
            wv3 = fy1 * fx1
            pltpu.make_async_copy(table.at[idxv.at[sb]], rows[sb],
                                  sems[sb]).wait()
            for g in range(_AG):
                w0 = wv0[g]
                w1 = wv1[g]
                w2 = wv2[g]
                w3 = wv3[g]
                for c in range(CV):
                    sl = pl.ds(c * L, L)
                    sr = pl.ds(C + c * L, L)
                    outv[pl.ds((sb * _AG + g) * C + c * L, L)] = (
                        rows[sb][g, sl] * w0 + rows[sb][g, sr] * w1
                        + rows[sb][L + g, sl] * w2 + rows[sb][L + g, sr] * w3)

            @pl.when(i + _ANB < _ACH)
            def _():
                issue(i + _ANB, sb)

        pltpu.sync_copy(outv,
                        out.at[pl.ds((base_pt + i2 * _ANB * _AG) * C,
                                     _ANB * _AG * C)])
        return carry

    lax.fori_loop(0, _ACH // _ANB, chunk2, 0)


_anchor_call = functools.partial(
    pl.kernel,
    out_type=jax.ShapeDtypeStruct((BN * C,), jnp.float32),
    mesh=_MESH,
    compiler_params=pltpu.CompilerParams(needs_layout_passes=False),
    scratch_types=[
        pltpu.VMEM((PPT,), jnp.float32),
        pltpu.VMEM((PPT,), jnp.float32),
        pltpu.VMEM((_ANB, 2 * L), jnp.int32),
        pltpu.VMEM((4 * L,), jnp.float32),
        [pltpu.VMEM((2 * L, C2), jnp.float32) for _ in range(_ANB)],
        pltpu.VMEM((_ANB * _AG * C,), jnp.float32),
        [pltpu.SemaphoreType.DMA for _ in range(_ANB)],
    ],
)


_BLKP = 1024


def _leaky(x):
    return jnp.where(x >= 0, x, 0.2 * x)


def _softplus(x):
    return jnp.maximum(x, 0.0) + jnp.log1p(jnp.exp(-jnp.abs(x)))


def _dot(a, b):
    return lax.dot_general(a, b, (((1,), (0,)), ((), ())),
                           preferred_element_type=jnp.float32,
                           precision=lax.Precision.HIGHEST)


def _mlp_body(fa_ref, co_ref, ce_ref, W1_ref, b1_ref, W2_ref, b2_ref,
              W3_ref, b3_ref, idx_ref, w_ref):
    fa = fa_ref[...]
    co = co_ref[...]
    ce = ce_ref[...]
    W1 = W1_ref[...]
    h = _dot(fa, W1[:C, :])
    h += co[:, 0:1] * W1[C:C + 1, :] + co[:, 1:2] * W1[C + 1:C + 2, :]
    h += ce[:, 0:1] * W1[C + 2:C + 3, :] + ce[:, 1:2] * W1[C + 3:C + 4, :]
    h += b1_ref[...]
    h = _leaky(h)
    h = _leaky(h + _dot(h, W2_ref[...]) + b2_ref[...])
    o = _dot(h, W3_ref[...]) + b3_ref[...]

    r = jnp.clip(_softplus(o[:, 0:1]) + 0.1, 0.1, 4.0)
    sigma = jnp.clip(_softplus(o[:, 1:2]) + 0.5, 0.5, 6.0)
    res18 = jnp.tanh(o[:, 2:2 + 2 * K]) * 0.5
    graw = o[:, 2 + 2 * K:2 + 3 * K]
    gate = 1.0 / (1.0 + jnp.exp(-graw))

    ii = lax.broadcasted_iota(jnp.int32, (1, 2 * K), 1)
    kk = ii // 2
    base18 = jnp.where(ii % 2 == 0, kk % 3 - 1, kk // 3 - 1).astype(jnp.float32)
    off18 = r * base18 + res18

    rows18 = lax.broadcasted_iota(jnp.int32, (2 * K, K), 0)
    cols18 = lax.broadcasted_iota(jnp.int32, (2 * K, K), 1)
    Mx = (rows18 == 2 * cols18).astype(jnp.float32)
    My = (rows18 == 2 * cols18 + 1).astype(jnp.float32)
    offx = _dot(off18, Mx)
    offy = _dot(off18, My)
    d2 = _dot(off18 * off18, Mx + My)

    sig_eff = sigma * 2.0
    w_geo = jnp.exp(-0.5 * d2 / (sig_eff * sig_eff + 1e-08))
    wk = w_geo * gate
    wn = wk / (jnp.sum(wk, axis=1, keepdims=True) + 1e-08)

    sx = 2.0 / (W - 1)
    sy = 2.0 / (H - 1)
    gx = jnp.clip((co[:, 0:1] + offx * sx + 1.0) * (0.5 * (W - 1)),
                  0.0, float(W - 1))
    gy = jnp.clip((co[:, 1:2] + offy * sy + 1.0) * (0.5 * (H - 1)),
                  0.0, float(H - 1))
    x0 = gx.astype(jnp.int32)
    y0 = gy.astype(jnp.int32)
    fx1 = gx - x0.astype(jnp.float32)
    fy1 = gy - y0.astype(jnp.float32)
    fx0 = 1.0 - fx1
    fy0 = 1.0 - fy1

    p = (pl.program_id(0) * _BLKP
         + lax.broadcasted_iota(jnp.int32, (_BLKP, 1), 0))
    bb = jnp.where(p >= N, HWp, 0).astype(jnp.int32)
    i00 = bb + y0 * W + x0
    i10 = jnp.minimum(i00 + W, BHW - 1)
    idx_ref[...] = jnp.concatenate([i00, i10], axis=1)
    w_ref[...] = jnp.concatenate([wn * fy0 * fx0, wn * fy0 * fx1,
                                  wn * fy1 * fx0, wn * fy1 * fx1], axis=1)


def _mlp_prep(f_anchor, co, ce, W1, b1, W2, b2, W3, b3):
    full = lambda i, j=None: (0, 0)
    grid = (BN // _BLKP,)
    return pl.pallas_call(
        _mlp_body,
        grid=grid,
        in_specs=[
            pl.BlockSpec((_BLKP, C), lambda i: (i, 0)),
            pl.BlockSpec((_BLKP, 2), lambda i: (i, 0)),
            pl.BlockSpec((_BLKP, 2), lambda i: (i, 0)),
            pl.BlockSpec((C + 4, HID), lambda i: (0, 0)),
            pl.BlockSpec((1, HID), lambda i: (0, 0)),
            pl.BlockSpec((HID, HID), lambda i: (0, 0)),
            pl.BlockSpec((1, HID), lambda i: (0, 0)),
            pl.BlockSpec((HID, 2 + 3 * K), lambda i: (0, 0)),
            pl.BlockSpec((1, 2 + 3 * K), lambda i: (0, 0)),
        ],
        out_specs=[
            pl.BlockSpec((_BLKP, 2 * K), lambda i: (i, 0)),
            pl.BlockSpec((_BLKP, J), lambda i: (i, 0)),
        ],
        out_shape=[
            jax.ShapeDtypeStruct((BN, 2 * K), jnp.int32),
            jax.ShapeDtypeStruct((BN, J), jnp.float32),
        ],
    )(f_anchor, co, ce, W1, b1, W2, b2, W3, b3)


_DG = 4
_CHN = PPT // _DG
_NBUF = 2
_JR = 2 * K


def _deform_body(table, idx_hbm, w_hbm, out, idxv, wtv, outv, rows, sems):
    wid = lax.axis_index("s") * NC + lax.axis_index("c")
    base_pt = wid * PPT
    pltpu.sync_copy(idx_hbm.at[pl.ds(wid * _CHN, _CHN)], idxv)
    pltpu.sync_copy(w_hbm.at[pl.ds(base_pt * J, PPT * J)],
                    wtv.at[pl.ds(0, PPT * J)])

    for b in range(_NBUF):
        pltpu.async_copy(table.at[idxv.at[b]], rows[b], sems[b])

    def chunk(ci, carry):
        for b in range(_NBUF):
            cc = ci * _NBUF + b
            pltpu.make_async_copy(table.at[idxv.at[0]], rows[b],
                                  sems[b]).wait()
            for g in range(_DG):
                wbase = (cc * _DG + g) * J
                wvecs = [wtv[pl.ds(wbase + v * L, L)] for v in range(3)]
                accs = [None] * CV
                for k in range(K):
                    w00 = wvecs[k // L][k % L]
                    w01 = wvecs[(K + k) // L][(K + k) % L]
                    w10 = wvecs[(2 * K + k) // L][(2 * K + k) % L]
                    w11 = wvecs[(3 * K + k) // L][(3 * K + k) % L]
                    rA = g * _JR + k
                    rB = g * _JR + K + k
                    for c in range(CV):
                        sl = pl.ds(c * L, L)
                        sr = pl.ds(C + c * L, L)
                        t = (rows[b][rA, sl] * w00 + rows[b][rA, sr] * w01
                             + rows[b][rB, sl] * w10 + rows[b][rB, sr] * w11)
                        accs[c] = t if k == 0 else accs[c] + t
                for c in range(CV):
                    outv[pl.ds(g * C + c * L, L)] = accs[c]
            nxt = cc + _NBUF

            @pl.when(nxt < _CHN)
            def _():
                pltpu.async_copy(table.at[idxv.at[nxt]], rows[b], sems[b])

            pltpu.sync_copy(outv, out.at[pl.ds((base_pt + cc * _DG) * C,
                                               _DG * C)])
        return carry

    lax.fori_loop(0, _CHN // _NBUF, chunk, 0)


_deform_call = functools.partial(
    pl.kernel,
    out_type=jax.ShapeDtypeStruct((BN * C,), jnp.float32),
    mesh=_MESH,
    compiler_params=pltpu.CompilerParams(needs_layout_passes=False),
    scratch_types=[
        pltpu.VMEM((_CHN, _DG * _JR), jnp.int32),
        pltpu.VMEM((PPT * J + L,), jnp.float32),
        pltpu.VMEM((_DG * C,), jnp.float32),
        [pltpu.VMEM((_DG * _JR, C2), jnp.float32) for _ in range(_NBUF)],
        [pltpu.SemaphoreType.DMA for _ in range(_NBUF)],
    ],
)


def kernel(feat_map, coords_2d, cell_2d, W1, b1, W2, b2, W3, b3):
    table = _make_table(feat_map)

    co = coords_2d.reshape(BN, 2)
    ce = cell_2d.reshape(BN, 2)
    xs = co[:, 0].reshape(BN)
    ys = co[:, 1].reshape(BN)

    f_anchor = _anchor_call(_anchor_body)(table, xs, ys).reshape(BN, C)

    idx_all, w_all = _mlp_prep(f_anchor, co, ce, W1,
                               b1.reshape(1, HID), W2, b2.reshape(1, HID),
                               W3, b3.reshape(1, 2 + 3 * K))

    out = _deform_call(_deform_body)(table,
                                     idx_all.reshape(BN // _DG, _DG * _JR),
                                     w_all.reshape(BN * J))
    return out.reshape(B, N, C)
